# Initial kernel scaffold; baseline (speedup 1.0000x reference)
import jax
import jax.numpy as jnp
from jax.experimental import pallas as pl


def kernel(x, edge_index, W, b):
    raise NotImplementedError("write your pallas kernel here")


# speedup vs baseline: 22.6891x; 22.6891x over previous
import functools

import jax
import jax.numpy as jnp
from jax import lax
from jax.experimental import pallas as pl
from jax.experimental.pallas import tpu as pltpu
from jax.experimental.pallas import tpu_sc as plsc

N = 10000
D = 128
E = 320000

NC = 2
NS = 16
NW = NC * NS
K = 128
NCHUNK = E // K
BASE_CH = NCHUNK // NW
REM_CH = NCHUNK % NW
NB = 80
NBLK = N // NB
NPAD = 10112
NBK = NPAD // 128
EPT = E // NW

_mesh = plsc.VectorSubcoreMesh(core_axis_name="c", subcore_axis_name="s")


@functools.partial(
    pl.kernel,
    out_type=jax.ShapeDtypeStruct((NC * NPAD,), jnp.float32),
    mesh=_mesh,
    compiler_params=pltpu.CompilerParams(needs_layout_passes=False),
    scratch_types=[
        pltpu.VMEM((EPT,), jnp.int32),
        pltpu.VMEM((NPAD,), jnp.float32),
        pltpu.VMEM((NS, 128), jnp.float32),
        pltpu.VMEM((128,), jnp.float32),
        pltpu.VMEM_SHARED((NS, NPAD), jnp.float32),
    ],
)
def _deg_kernel(row_hbm, deg_hbm, idx_v, deg_v, red_v, res_v, slab_sh):
    c = lax.axis_index("c")
    s = lax.axis_index("s")
    wid = c * NS + s

    def fz(i, _):
        deg_v[pl.ds(i * 16, 16)] = jnp.zeros((16,), jnp.float32)
        return 0

    lax.fori_loop(0, NPAD // 16, fz, 0)

    pltpu.sync_copy(row_hbm.at[pl.ds(wid * EPT, EPT)], idx_v)
    ones = jnp.ones((16,), jnp.float32)

    def acc(i, _):
        ivec = idx_v[pl.ds(i * 16, 16)]
        plsc.addupdate_scatter(deg_v, [ivec], ones)
        return 0

    lax.fori_loop(0, EPT // 16, acc, 0)

    pltpu.sync_copy(deg_v, slab_sh.at[s])
    plsc.subcore_barrier()

    nbk = jnp.where(s < NBK % NS, NBK // NS + 1, NBK // NS)

    def red(k, _):
        b = s + NS * k
        pltpu.sync_copy(slab_sh.at[:, pl.ds(b * 128, 128)], red_v)

        def col(j, _2):
            def rowadd(t, a):
                return a + red_v[t, pl.ds(j * 16, 16)]

            res_v[pl.ds(j * 16, 16)] = lax.fori_loop(
                0, NS, rowadd, jnp.zeros((16,), jnp.float32))
            return 0

        lax.fori_loop(0, 8, col, 0)
        pltpu.sync_copy(res_v, deg_hbm.at[pl.ds(c * NPAD + b * 128, 128)])
        return 0

    lax.fori_loop(0, nbk, red, 0)


@functools.partial(
    pl.kernel,
    out_type=jax.ShapeDtypeStruct((NC, N, D), jnp.float32),
    mesh=_mesh,
    scratch_types=[
        pltpu.VMEM((K,), jnp.int32),
        pltpu.VMEM((K,), jnp.int32),
        pltpu.VMEM((K, D), jnp.float32),
        pltpu.VMEM((NB, D), jnp.float32),
        pltpu.VMEM_SHARED((N, D), jnp.float32),
        pltpu.SemaphoreType.DMA,
    ],
)
def _msg_kernel(y_hbm, col_hbm, row_hbm, h_hbm, colv, rowv, rows_v, zb_v,
                acc_sh, sem):
    c = lax.axis_index("c")
    s = lax.axis_index("s")
    wid = c * NS + s
    nblk = jnp.where(s < NBLK % NS, NBLK // NS + 1, NBLK // NS)

    def fill_zero(t, _):
        i = t // 8
        j = t % 8
        zb_v[i, pl.ds(j * 16, 16)] = jnp.zeros((16,), jnp.float32)
        return 0

    lax.fori_loop(0, NB * 8, fill_zero, 0)

    def zero_acc(k, _):
        pltpu.sync_copy(zb_v, acc_sh.at[pl.ds((s + NS * k) * NB, NB)])
        return 0

    lax.fori_loop(0, nblk, zero_acc, 0)
    plsc.subcore_barrier()

    nch = jnp.where(wid < REM_CH, BASE_CH + 1, BASE_CH)

    def body(k, _):
        ch = wid + NW * k
        base = ch * K
        pltpu.sync_copy(col_hbm.at[pl.ds(base, K)], colv)
        pltpu.async_copy(y_hbm.at[colv], rows_v, sem).wait()
        pltpu.sync_copy(row_hbm.at[pl.ds(base, K)], rowv)
        pltpu.sync_copy(rows_v, acc_sh.at[rowv], add=True)
        return 0

    lax.fori_loop(0, nch, body, 0)
    plsc.subcore_barrier()

    def dump(k, _):
        base = (s + NS * k) * NB
        pltpu.sync_copy(acc_sh.at[pl.ds(base, NB)], zb_v)
        pltpu.sync_copy(zb_v, h_hbm.at[c, pl.ds(base, NB)])
        return 0

    lax.fori_loop(0, nblk, dump, 0)


def _dis_from_deg(deg_ref):
    deg = deg_ref[0, :N] + deg_ref[1, :N]
    return jnp.where(deg > 0, lax.rsqrt(jnp.maximum(deg, 1.0)), 0.0)


def _scale_body(deg_ref, x_ref, y_ref):
    dis = _dis_from_deg(deg_ref)
    y_ref[...] = x_ref[...] * dis[:, None]


_scale_call = pl.pallas_call(
    _scale_body,
    out_shape=jax.ShapeDtypeStruct((N, D), jnp.float32),
)


def _out_body(hp_ref, deg_ref, w_ref, b_ref, o_ref):
    dis = _dis_from_deg(deg_ref)
    h = (hp_ref[0] + hp_ref[1]) * dis[:, None]
    z = jnp.dot(h, w_ref[...], preferred_element_type=jnp.float32)
    z = z + b_ref[...]
    o_ref[...] = jnp.where(z >= 0, z, 0.2 * z)


_out_call = pl.pallas_call(
    _out_body,
    out_shape=jax.ShapeDtypeStruct((N, D), jnp.float32),
)


def kernel(x, edge_index, W, b):
    ei = edge_index.astype(jnp.int32)
    row = ei[0]
    col = ei[1]
    deg_p = _deg_kernel(row).reshape(NC, NPAD)
    y = _scale_call(deg_p, x)
    h_p = _msg_kernel(y, col, row)
    return _out_call(h_p, deg_p, W, b.reshape(1, D))
